# Initial kernel scaffold; baseline (speedup 1.0000x reference)
import jax
import jax.numpy as jnp
from jax.experimental import pallas as pl


def kernel(x, edge_index, W_embed, b_embed, W1, b1, W2, b2, Wc, bc):
    raise NotImplementedError("write your pallas kernel here")


# speedup vs baseline: 17.9024x; 17.9024x over previous
import functools

import jax
import jax.numpy as jnp
from jax import lax
from jax.experimental import pallas as pl
from jax.experimental.pallas import tpu as pltpu
from jax.experimental.pallas import tpu_sc as plsc

N = 10000
E = 320000
IN_DIM = 128
HID = 64
OUT = 64

NP_ = 10240
NW = 32
CHUNK = 128
CH = 79
EPT = CH * CHUNK
E_PAD = NW * EPT
ROWS_PER_TILE = NP_ // 16

_mesh = plsc.VectorSubcoreMesh(core_axis_name="c", subcore_axis_name="s")


def _deg_body(dst_hbm, out_hbm, didx, hist):
    cid = lax.axis_index("c")
    sid = lax.axis_index("s")
    w = cid * 16 + sid

    pltpu.sync_copy(dst_hbm.at[w], didx)

    zeros16 = jnp.zeros((16,), jnp.float32)

    def zero_body(i, carry):
        hist[i, :] = zeros16
        return carry

    lax.fori_loop(0, NP_ // 16, zero_body, 0)

    ones16 = jnp.ones((16,), jnp.float32)

    def acc_body(i, carry):
        idxv = didx[pl.ds(i * 16, 16)]
        row = lax.shift_right_logical(idxv, 4)
        col = lax.bitwise_and(idxv, 15)
        plsc.addupdate_scatter(hist, [row, col], ones16)
        return carry

    lax.fori_loop(0, EPT // 16, acc_body, 0)

    pltpu.sync_copy(hist, out_hbm.at[w])


_deg_kernel = functools.partial(
    pl.kernel,
    out_type=jax.ShapeDtypeStruct((NW, NP_ // 16, 16), jnp.float32),
    mesh=_mesh,
    compiler_params=pltpu.CompilerParams(needs_layout_passes=False),
    scratch_types=[
        pltpu.VMEM((EPT,), jnp.int32),
        pltpu.VMEM((NP_ // 16, 16), jnp.float32),
    ],
)(_deg_body)


def _prop_body(g_hbm, src_hbm, dst_hbm, z_hbm, out_hbm, sidx, didx, rows, accum):
    cid = lax.axis_index("c")
    sid = lax.axis_index("s")
    w = cid * 16 + sid

    pltpu.sync_copy(z_hbm.at[pl.ds(sid * ROWS_PER_TILE, ROWS_PER_TILE)],
                    accum.at[pl.ds(sid * ROWS_PER_TILE, ROWS_PER_TILE)])
    pltpu.sync_copy(src_hbm.at[w], sidx)
    pltpu.sync_copy(dst_hbm.at[w], didx)
    plsc.subcore_barrier()

    def body(j, carry):
        pltpu.sync_copy(g_hbm.at[sidx.at[j]], rows)
        pltpu.sync_copy(rows, accum.at[didx.at[j]], add=True)
        return carry

    lax.fori_loop(0, CH, body, 0)
    plsc.subcore_barrier()

    pltpu.sync_copy(accum.at[pl.ds(sid * ROWS_PER_TILE, ROWS_PER_TILE)],
                    out_hbm.at[cid, pl.ds(sid * ROWS_PER_TILE, ROWS_PER_TILE)])


_prop_kernel = functools.partial(
    pl.kernel,
    out_type=jax.ShapeDtypeStruct((2, NP_, HID), jnp.float32),
    mesh=_mesh,
    compiler_params=pltpu.CompilerParams(needs_layout_passes=False,
                                         use_tc_tiling_on_sc=False),
    scratch_types=[
        pltpu.VMEM((CH, CHUNK), jnp.int32),
        pltpu.VMEM((CH, CHUNK), jnp.int32),
        pltpu.VMEM((CHUNK, HID), jnp.float32),
        pltpu.VMEM_SHARED((NP_, HID), jnp.float32),
    ],
)(_prop_body)


def _dinv_block(degt):
    dsum = jnp.sum(degt, axis=1, keepdims=True)
    dinv = lax.rsqrt(dsum + 1.0)
    rows = lax.broadcasted_iota(jnp.int32, (NP_, 1), 0)
    return jnp.where(rows < N, dinv, 0.0)


def _tc1_body(x_ref, wet_ref, be_ref, w1t_ref, degt_ref, h_ref, g1_ref):
    h = jnp.dot(x_ref[...], wet_ref[...], preferred_element_type=jnp.float32)
    h = jnp.maximum(h + be_ref[...], 0.0)
    h_ref[...] = h
    dinv = _dinv_block(degt_ref[...])
    hw = jnp.dot(h, w1t_ref[...], preferred_element_type=jnp.float32)
    g1_ref[...] = hw * dinv


def _tc2_body(pa_ref, pb_ref, g1_ref, degt_ref, b1_ref, w2t_ref, h1_ref, g2_ref):
    dinv = _dinv_block(degt_ref[...])
    s = pa_ref[...] + pb_ref[...] + g1_ref[...]
    h1 = jnp.maximum(s * dinv + b1_ref[...], 0.0)
    h1_ref[...] = h1
    hw = jnp.dot(h1, w2t_ref[...], preferred_element_type=jnp.float32)
    g2_ref[...] = hw * dinv


def _tc3_body(pa_ref, pb_ref, g2_ref, degt_ref, b2_ref, h_ref, h1_ref,
              wca_ref, wcb_ref, wcc_ref, bc_ref, out_ref):
    dinv = _dinv_block(degt_ref[...])
    s = pa_ref[...] + pb_ref[...] + g2_ref[...]
    h2 = jnp.maximum(s * dinv + b2_ref[...], 0.0)
    out = jnp.dot(h_ref[...], wca_ref[...], preferred_element_type=jnp.float32)
    out = out + jnp.dot(h1_ref[...], wcb_ref[...], preferred_element_type=jnp.float32)
    out = out + jnp.dot(h2, wcc_ref[...], preferred_element_type=jnp.float32)
    out_ref[...] = out + bc_ref[...]


def _tc_call(body, n_out):
    return pl.pallas_call(
        body,
        out_shape=[jax.ShapeDtypeStruct((NP_, HID), jnp.float32)] * n_out,
    )


def kernel(x, edge_index, W_embed, b_embed, W1, b1, W2, b2, Wc, bc):
    f32 = jnp.float32
    src = edge_index[0]
    dst = edge_index[1]
    pad = E_PAD - E
    srcp = jnp.concatenate([src, jnp.full((pad,), N, jnp.int32)])
    dstp = jnp.concatenate([dst, jnp.full((pad,), N, jnp.int32)])
    src3 = srcp.reshape(NW, CH, CHUNK)
    dst3 = dstp.reshape(NW, CH, CHUNK)
    dstf = dstp.reshape(NW, EPT)

    xp = jnp.zeros((NP_, IN_DIM), f32).at[:N].set(x)
    zeros2d = jnp.zeros((NP_, HID), f32)

    wet = W_embed.T.astype(f32)
    w1t = W1.T.astype(f32)
    w2t = W2.T.astype(f32)
    wca = Wc[:, :HID].T.astype(f32)
    wcb = Wc[:, HID:2 * HID].T.astype(f32)
    wcc = Wc[:, 2 * HID:].T.astype(f32)
    be = b_embed.reshape(1, HID)
    b1r = b1.reshape(1, HID)
    b2r = b2.reshape(1, HID)
    bcr = bc.reshape(1, HID)

    degp = _deg_kernel(dstf).reshape(NW, NP_)
    degt = degp.T

    h, g1 = _tc_call(_tc1_body, 2)(xp, wet, be, w1t, degt)

    p1 = _prop_kernel(g1, src3, dst3, zeros2d)

    h1, g2 = _tc_call(_tc2_body, 2)(p1[0], p1[1], g1, degt, b1r, w2t)

    p2 = _prop_kernel(g2, src3, dst3, zeros2d)

    (out,) = _tc_call(_tc3_body, 1)(p2[0], p2[1], g2, degt, b2r, h, h1,
                                    wca, wcb, wcc, bcr)
    return out[:N]
